# Initial kernel scaffold; baseline (speedup 1.0000x reference)
import jax
import jax.numpy as jnp
from jax.experimental import pallas as pl


def kernel(x, edge_index, batch, W1, W2):
    raise NotImplementedError("write your pallas kernel here")


# speedup vs baseline: 6.0421x; 6.0421x over previous
import functools

import jax
import jax.numpy as jnp
from jax import lax
from jax.experimental import pallas as pl
from jax.experimental.pallas import tpu as pltpu
from jax.experimental.pallas import tpu_sc as plsc

_N = 10000
_NPAD = 10240
_E = 320000
_D = 128
_NC = 2
_NS = 16
_L = 16

@functools.cache
def _sc_mesh():
    return plsc.VectorSubcoreMesh(
        core_axis_name="c", subcore_axis_name="s", num_cores=_NC, num_subcores=_NS
    )


_EW1 = _E // (_NC * _NS)
_CH1 = 80
_NCH1 = _EW1 // _CH1
_RPS = _NPAD // _NS


def _agg_rows_body(x_hbm, z_hbm, src_hbm, dst_hbm, out_hbm, src_v, dst_v, rows_v, acc, sem):
    c = lax.axis_index("c")
    s = lax.axis_index("s")
    wid = c * _NS + s
    pltpu.sync_copy(z_hbm.at[pl.ds(s * _RPS, _RPS)], acc.at[pl.ds(s * _RPS, _RPS)])
    plsc.subcore_barrier()
    base = wid * _EW1

    def step(i, carry):
        off = base + i * _CH1
        pltpu.sync_copy(src_hbm.at[pl.ds(off, _CH1)], src_v)
        pltpu.sync_copy(dst_hbm.at[pl.ds(off, _CH1)], dst_v)
        pltpu.async_copy(x_hbm.at[src_v], rows_v, sem).wait()
        pltpu.sync_copy(rows_v, acc.at[dst_v], add=True)
        return carry

    lax.fori_loop(0, _NCH1, step, 0)
    plsc.subcore_barrier()
    pltpu.sync_copy(acc.at[pl.ds(s * _RPS, _RPS)], out_hbm.at[c, pl.ds(s * _RPS, _RPS)])


@functools.cache
def _agg_rows():
    return pl.kernel(
        _agg_rows_body,
        out_type=jax.ShapeDtypeStruct((_NC, _NPAD, _D), jnp.float32),
        mesh=_sc_mesh(),
        scratch_types=[
            pltpu.VMEM((_CH1,), jnp.int32),
            pltpu.VMEM((_CH1,), jnp.int32),
            pltpu.VMEM((_CH1, _D), jnp.float32),
            pltpu.VMEM_SHARED((_NPAD, _D), jnp.float32),
            pltpu.SemaphoreType.DMA,
        ],
    )

_RB = 2048


def _mlp_body(p_ref, w1_ref, w2_ref, g_ref):
    a = p_ref[0] + p_ref[1]
    h1 = jnp.maximum(
        jnp.dot(a, w1_ref[...], preferred_element_type=jnp.float32), 0.0
    )
    g_ref[...] = jnp.dot(h1, w2_ref[...], preferred_element_type=jnp.float32)


def _mlp(p, W1, W2):
    return pl.pallas_call(
        _mlp_body,
        grid=(_NPAD // _RB,),
        in_specs=[
            pl.BlockSpec((2, _RB, _D), lambda i: (0, i, 0)),
            pl.BlockSpec((_D, _D), lambda i: (0, 0)),
            pl.BlockSpec((_D, 1), lambda i: (0, 0)),
        ],
        out_specs=pl.BlockSpec((_RB, 1), lambda i: (i, 0)),
        out_shape=jax.ShapeDtypeStruct((_NPAD, 1), jnp.float32),
    )(p, W1, W2)


_NP = 10240
_EW2 = _E // _NS
_CH2 = 80
_NCH2 = _EW2 // _CH2
_PPS = _NP // _NS


def _agg_scalar_body(g_hbm, z2_hbm, src_hbm, dst_hbm, out_hbm, src_v, dst_v, msg_v, vbuf, acc2, sem):
    c = lax.axis_index("c")
    s = lax.axis_index("s")

    @pl.when(c == 0)
    def _():
        pltpu.sync_copy(z2_hbm.at[pl.ds(s * _PPS, _PPS)], acc2.at[pl.ds(s * _PPS, _PPS)])
        plsc.subcore_barrier()
        base = s * _EW2

        def step(i, carry):
            off = base + i * _CH2
            pltpu.sync_copy(src_hbm.at[pl.ds(off, _CH2)], src_v)
            pltpu.sync_copy(dst_hbm.at[pl.ds(off, _CH2)], dst_v)
            pltpu.async_copy(g_hbm.at[src_v], msg_v, sem).wait()
            pltpu.sync_copy(msg_v, acc2.at[dst_v], add=True)
            return carry

        lax.fori_loop(0, _NCH2, step, 0)
        plsc.subcore_barrier()
        pltpu.sync_copy(acc2.at[pl.ds(s * _PPS, _PPS)], vbuf)

        def relu_step(j, carry):
            vbuf[pl.ds(j * _L, _L)] = jnp.maximum(vbuf[pl.ds(j * _L, _L)], 0.0)
            return carry

        lax.fori_loop(0, _PPS // _L, relu_step, 0)
        pltpu.sync_copy(vbuf, out_hbm.at[pl.ds(s * _PPS, _PPS)])


@functools.cache
def _agg_scalar():
    return pl.kernel(
        _agg_scalar_body,
        out_type=jax.ShapeDtypeStruct((_NP,), jnp.float32),
        mesh=_sc_mesh(),
        scratch_types=[
            pltpu.VMEM((_CH2,), jnp.int32),
            pltpu.VMEM((_CH2,), jnp.int32),
            pltpu.VMEM((_CH2,), jnp.float32),
            pltpu.VMEM((_PPS,), jnp.float32),
            pltpu.VMEM_SHARED((_NP,), jnp.float32),
            pltpu.SemaphoreType.DMA,
        ],
    )


@jax.jit
def kernel(x, edge_index, batch, W1, W2):
    del batch
    src = edge_index[0]
    dst = edge_index[1]
    z = jnp.zeros((_NPAD, _D), jnp.float32)
    p = _agg_rows()(x, z, src, dst)
    g = _mlp(p, W1, W2)
    z2 = jnp.zeros((_NP,), jnp.float32)
    o = _agg_scalar()(g.reshape(_NP), z2, src, dst)
    return o[:_N].reshape(_N, 1)


# speedup vs baseline: 19.7550x; 3.2696x over previous
import functools

import jax
import jax.numpy as jnp
from jax import lax
from jax.experimental import pallas as pl
from jax.experimental.pallas import tpu as pltpu
from jax.experimental.pallas import tpu_sc as plsc

_N = 10000
_NPAD = 10240
_E = 320000
_D = 128
_NC = 2
_NS = 16
_L = 16


@functools.cache
def _sc_mesh():
    return plsc.VectorSubcoreMesh(
        core_axis_name="c", subcore_axis_name="s", num_cores=_NC, num_subcores=_NS
    )


_CH = 80
_NW1 = _NC * _NS
_EW1 = _E // _NW1
_CPW1 = _EW1 // _CH
_RPS = _NPAD // _NS


def _agg_rows_body(x_hbm, z_hbm, src_hbm, dst_hbm, out_hbm,
                   srcall, dstall, srcc, dstc, rows, acc, gs0, gs1, ss0, ss1):
    c = lax.axis_index("c")
    s = lax.axis_index("s")
    wid = c * _NS + s
    pltpu.sync_copy(z_hbm.at[pl.ds(s * _RPS, _RPS)], acc.at[pl.ds(s * _RPS, _RPS)])
    pltpu.sync_copy(src_hbm.at[pl.ds(wid * _EW1, _EW1)], srcall)
    pltpu.sync_copy(dst_hbm.at[pl.ds(wid * _EW1, _EW1)], dstall)
    plsc.subcore_barrier()

    def chunk(j, b, gsem, ssem, first):
        if not first:
            pltpu.make_async_copy(rows.at[b], acc.at[dstc.at[b]], ssem).wait()
        for k in range(_CH // _L):
            srcc[b, pl.ds(k * _L, _L)] = srcall[pl.ds(j * _CH + k * _L, _L)]
            dstc[b, pl.ds(k * _L, _L)] = dstall[pl.ds(j * _CH + k * _L, _L)]
        pltpu.async_copy(x_hbm.at[srcc.at[b]], rows.at[b], gsem).wait()
        pltpu.async_copy(rows.at[b], acc.at[dstc.at[b]], ssem, add=True)

    def pair(g, carry):
        chunk(2 * g, 0, gs0, ss0, False)
        chunk(2 * g + 1, 1, gs1, ss1, False)
        return carry

    chunk(0, 0, gs0, ss0, True)
    chunk(1, 1, gs1, ss1, True)
    lax.fori_loop(1, _CPW1 // 2, pair, 0)
    chunk(_CPW1 - 1, 0, gs0, ss0, False)
    pltpu.make_async_copy(rows.at[0], acc.at[dstc.at[0]], ss0).wait()
    pltpu.make_async_copy(rows.at[1], acc.at[dstc.at[1]], ss1).wait()

    plsc.subcore_barrier()
    pltpu.sync_copy(acc.at[pl.ds(s * _RPS, _RPS)], out_hbm.at[c, pl.ds(s * _RPS, _RPS)])


@functools.cache
def _agg_rows():
    return pl.kernel(
        _agg_rows_body,
        out_type=jax.ShapeDtypeStruct((_NC, _NPAD, _D), jnp.float32),
        mesh=_sc_mesh(),
        scratch_types=[
            pltpu.VMEM((_EW1,), jnp.int32),
            pltpu.VMEM((_EW1,), jnp.int32),
            pltpu.VMEM((2, _CH), jnp.int32),
            pltpu.VMEM((2, _CH), jnp.int32),
            pltpu.VMEM((2, _CH, _D), jnp.float32),
            pltpu.VMEM_SHARED((_NPAD, _D), jnp.float32),
            pltpu.SemaphoreType.DMA,
            pltpu.SemaphoreType.DMA,
            pltpu.SemaphoreType.DMA,
            pltpu.SemaphoreType.DMA,
        ],
    )


_RB = 2048


def _mlp_body(p_ref, w1_ref, w2_ref, g_ref):
    a = p_ref[0] + p_ref[1]
    h1 = jnp.maximum(
        jnp.dot(a, w1_ref[...], preferred_element_type=jnp.float32), 0.0
    )
    g_ref[...] = jnp.dot(h1, w2_ref[...], preferred_element_type=jnp.float32)


def _mlp(p, W1, W2):
    return pl.pallas_call(
        _mlp_body,
        grid=(_NPAD // _RB,),
        in_specs=[
            pl.BlockSpec((2, _RB, _D), lambda i: (0, i, 0)),
            pl.BlockSpec((_D, _D), lambda i: (0, 0)),
            pl.BlockSpec((_D, 1), lambda i: (0, 0)),
        ],
        out_specs=pl.BlockSpec((_RB, 1), lambda i: (i, 0)),
        out_shape=jax.ShapeDtypeStruct((_NPAD, 1), jnp.float32),
    )(p, W1, W2)


_CPW2 = _E // (_NS * _CH)
_PPS = _NPAD // _NS


def _agg_scalar_body(g_hbm, z2_hbm, src_hbm, dst_hbm, out_hbm,
                     gtab, srcv, dstv, msgv, vbuf, acc2, ss0, ss1):
    c = lax.axis_index("c")
    s = lax.axis_index("s")

    @pl.when(c == 0)
    def _():
        pltpu.sync_copy(z2_hbm.at[pl.ds(s * _PPS, _PPS)], acc2.at[pl.ds(s * _PPS, _PPS)])
        pltpu.sync_copy(g_hbm, gtab)
        pltpu.sync_copy(src_hbm.at[s], srcv)
        pltpu.sync_copy(dst_hbm.at[s], dstv)
        plsc.subcore_barrier()

        def chunk(j, b, ssem, first):
            if not first:
                pltpu.make_async_copy(msgv.at[b], acc2.at[dstv.at[j]], ssem).wait()
            for k in range(_CH // _L):
                idx = srcv[j, pl.ds(k * _L, _L)]
                msgv[b, pl.ds(k * _L, _L)] = plsc.load_gather(gtab, [idx])
            pltpu.async_copy(msgv.at[b], acc2.at[dstv.at[j]], ssem, add=True)

        def pair(g, carry):
            chunk(2 * g, 0, ss0, False)
            chunk(2 * g + 1, 1, ss1, False)
            return carry

        chunk(0, 0, ss0, True)
        chunk(1, 1, ss1, True)
        lax.fori_loop(1, _CPW2 // 2, pair, 0)
        pltpu.make_async_copy(msgv.at[0], acc2.at[dstv.at[_CPW2 - 2]], ss0).wait()
        pltpu.make_async_copy(msgv.at[1], acc2.at[dstv.at[_CPW2 - 1]], ss1).wait()

        plsc.subcore_barrier()
        pltpu.sync_copy(acc2.at[pl.ds(s * _PPS, _PPS)], vbuf)

        def relu_step(j, carry):
            vbuf[pl.ds(j * _L, _L)] = jnp.maximum(vbuf[pl.ds(j * _L, _L)], 0.0)
            return carry

        lax.fori_loop(0, _PPS // _L, relu_step, 0)
        pltpu.sync_copy(vbuf, out_hbm.at[pl.ds(s * _PPS, _PPS)])


@functools.cache
def _agg_scalar():
    return pl.kernel(
        _agg_scalar_body,
        out_type=jax.ShapeDtypeStruct((_NPAD,), jnp.float32),
        mesh=_sc_mesh(),
        compiler_params=pltpu.CompilerParams(needs_layout_passes=False),
        scratch_types=[
            pltpu.VMEM((_NPAD,), jnp.float32),
            pltpu.VMEM((_CPW2, _CH), jnp.int32),
            pltpu.VMEM((_CPW2, _CH), jnp.int32),
            pltpu.VMEM((2, _CH), jnp.float32),
            pltpu.VMEM((_PPS,), jnp.float32),
            pltpu.VMEM_SHARED((_NPAD,), jnp.float32),
            pltpu.SemaphoreType.DMA,
            pltpu.SemaphoreType.DMA,
        ],
    )


@jax.jit
def kernel(x, edge_index, batch, W1, W2):
    del batch
    src = edge_index[0]
    dst = edge_index[1]
    z = jnp.zeros((_NPAD, _D), jnp.float32)
    p = _agg_rows()(x, z, src, dst)
    g = _mlp(p, W1, W2)
    src2 = src.reshape(_NS, _CPW2, _CH)
    dst2 = dst.reshape(_NS, _CPW2, _CH)
    z2 = jnp.zeros((_NPAD,), jnp.float32)
    o = _agg_scalar()(g.reshape(_NPAD), z2, src2, dst2)
    return o[:_N].reshape(_N, 1)
